# baseline (speedup 1.0000x reference)
import functools
import jax
import jax.numpy as jnp
from jax import lax
from jax.experimental import pallas as pl
from jax.experimental.pallas import tpu as pltpu, tpu_sc as plsc

_SEQ, _DIM = 8192, 1024
_SCALE = _DIM ** (-0.5)
_NC, _NS = 2, 16
_NW = _NC * _NS
_CH_ROWS = 16
_NBUF = 7
_CH_E = _CH_ROWS * _DIM

_SC_ROWS = 2048
_TC_ROWS = _SEQ - _SC_ROWS
_TC_BLOCK = 2048

_mesh = plsc.VectorSubcoreMesh(core_axis_name="c", subcore_axis_name="s")


def _make_sc_scale(total_rows, row_offset):
    rows_w = total_rows // _NW
    nchunk = rows_w // _CH_ROWS

    @functools.partial(
        pl.kernel,
        out_type=jax.ShapeDtypeStruct((total_rows, _DIM), jnp.float32),
        mesh=_mesh,
        scratch_types=[pltpu.VMEM((_NBUF * _CH_ROWS, _DIM), jnp.float32)]
        + [pltpu.SemaphoreType.DMA] * (2 * _NBUF),
    )
    def sc_scale(emb_hbm, out_hbm, buf, *sems):
        in_sems = sems[:_NBUF]
        out_sems = sems[_NBUF:]
        wid = lax.axis_index("s") * _NC + lax.axis_index("c")
        src_base = row_offset + wid * rows_w
        dst_base = wid * rows_w
        in_d = [None] * nchunk
        out_d = [None] * nchunk
        for i in range(min(_NBUF, nchunk)):
            in_d[i] = pltpu.async_copy(
                emb_hbm.at[pl.ds(src_base + i * _CH_ROWS, _CH_ROWS)],
                buf.at[pl.ds(i * _CH_ROWS, _CH_ROWS)],
                in_sems[i],
            )
        for i in range(nchunk):
            b = i % _NBUF
            in_d[i].wait()
            row0 = b * _CH_ROWS

            @plsc.parallel_loop(0, _CH_E, 16, unroll=8)
            def _(j):
                r = row0 + (j >> 10)
                c = pl.multiple_of(j & (_DIM - 1), 16)
                buf[r, pl.ds(c, 16)] = buf[r, pl.ds(c, 16)] * _SCALE

            out_d[i] = pltpu.async_copy(
                buf.at[pl.ds(row0, _CH_ROWS)],
                out_hbm.at[pl.ds(dst_base + i * _CH_ROWS, _CH_ROWS)],
                out_sems[b],
            )
            nxt = i + _NBUF
            if nxt < nchunk:
                out_d[i].wait()
                in_d[nxt] = pltpu.async_copy(
                    emb_hbm.at[pl.ds(src_base + nxt * _CH_ROWS, _CH_ROWS)],
                    buf.at[pl.ds(row0, _CH_ROWS)],
                    in_sems[b],
                )
        for i in range(max(0, nchunk - _NBUF), nchunk):
            out_d[i].wait()

    return sc_scale


_sc_scale = _make_sc_scale(_SC_ROWS, _TC_ROWS)


def _tc_body(emb_ref, o_ref):
    o_ref[...] = emb_ref[...] * _SCALE


def _tc_scale(emb):
    return pl.pallas_call(
        _tc_body,
        grid=(_TC_ROWS // _TC_BLOCK,),
        in_specs=[pl.BlockSpec((_TC_BLOCK, _DIM), lambda i: (i, 0))],
        out_specs=pl.BlockSpec((_TC_BLOCK, _DIM), lambda i: (i, 0)),
        out_shape=jax.ShapeDtypeStruct((_TC_ROWS, _DIM), emb.dtype),
    )(emb)


def kernel(x, emb):
    del x
    top = _tc_scale(emb)
    bottom = _sc_scale(emb)
    return jnp.concatenate([top, bottom], axis=0)


# speedup vs baseline: 1.3446x; 1.3446x over previous
import functools
import jax
import jax.numpy as jnp
from jax import lax
from jax.experimental import pallas as pl
from jax.experimental.pallas import tpu as pltpu, tpu_sc as plsc

_SEQ, _DIM = 8192, 1024
_SCALE = _DIM ** (-0.5)
_NC, _NS = 2, 16
_NW = _NC * _NS
_ROWS_W = _SEQ // _NW
_CH_ROWS = 16
_NCHUNK = _ROWS_W // _CH_ROWS
_NBUF = 3
_CH_E = _CH_ROWS * _DIM

_mesh = plsc.VectorSubcoreMesh(core_axis_name="c", subcore_axis_name="s")


@functools.partial(
    pl.kernel,
    out_type=jax.ShapeDtypeStruct((_SEQ, _DIM), jnp.float32),
    mesh=_mesh,
    scratch_types=[
        pltpu.VMEM((_NBUF * _CH_ROWS, _DIM), jnp.float32),
        pltpu.VMEM((_NBUF * _CH_ROWS, _DIM), jnp.float32),
    ]
    + [pltpu.SemaphoreType.DMA] * (2 * _NBUF),
)
def _sc_scale(emb_hbm, out_hbm, ibuf, obuf, *sems):
    in_sems = sems[:_NBUF]
    out_sems = sems[_NBUF:]
    wid = lax.axis_index("s") * _NC + lax.axis_index("c")
    base = wid * _ROWS_W
    in_d = [None] * _NCHUNK
    out_d = [None] * _NCHUNK
    for i in range(min(_NBUF, _NCHUNK)):
        in_d[i] = pltpu.async_copy(
            emb_hbm.at[pl.ds(base + i * _CH_ROWS, _CH_ROWS)],
            ibuf.at[pl.ds(i * _CH_ROWS, _CH_ROWS)],
            in_sems[i],
        )
    for i in range(_NCHUNK):
        b = i % _NBUF
        row0 = b * _CH_ROWS
        in_d[i].wait()
        if i >= _NBUF:
            out_d[i - _NBUF].wait()

        @plsc.parallel_loop(0, _CH_E, 16, unroll=8)
        def _(j):
            r = row0 + (j >> 10)
            c = pl.multiple_of(j & (_DIM - 1), 16)
            obuf[r, pl.ds(c, 16)] = ibuf[r, pl.ds(c, 16)] * _SCALE

        out_d[i] = pltpu.async_copy(
            obuf.at[pl.ds(row0, _CH_ROWS)],
            out_hbm.at[pl.ds(base + i * _CH_ROWS, _CH_ROWS)],
            out_sems[b],
        )
        nxt = i + _NBUF
        if nxt < _NCHUNK:
            in_d[nxt] = pltpu.async_copy(
                emb_hbm.at[pl.ds(base + nxt * _CH_ROWS, _CH_ROWS)],
                ibuf.at[pl.ds(row0, _CH_ROWS)],
                in_sems[b],
            )
    for i in range(max(0, _NCHUNK - _NBUF), _NCHUNK):
        out_d[i].wait()


def kernel(x, emb):
    del x
    return _sc_scale(emb)


# speedup vs baseline: 1.3465x; 1.0014x over previous
import functools
import jax
import jax.numpy as jnp
from jax import lax
from jax.experimental import pallas as pl
from jax.experimental.pallas import tpu as pltpu, tpu_sc as plsc

_SEQ, _DIM = 8192, 1024
_SCALE = _DIM ** (-0.5)
_NC, _NS = 2, 16
_NW = _NC * _NS
_ROWS_W = _SEQ // _NW
_CH_ROWS = 8
_NCHUNK = _ROWS_W // _CH_ROWS
_NBUF = 14
_CH_E = _CH_ROWS * _DIM

_mesh = plsc.VectorSubcoreMesh(core_axis_name="c", subcore_axis_name="s")


@functools.partial(
    pl.kernel,
    out_type=jax.ShapeDtypeStruct((_SEQ, _DIM), jnp.float32),
    mesh=_mesh,
    scratch_types=[pltpu.VMEM((_NBUF * _CH_ROWS, _DIM), jnp.float32)]
    + [pltpu.SemaphoreType.DMA] * (2 * _NBUF),
)
def _sc_scale(emb_hbm, out_hbm, buf, *sems):
    in_sems = sems[:_NBUF]
    out_sems = sems[_NBUF:]
    wid = lax.axis_index("s") * _NC + lax.axis_index("c")
    base = wid * _ROWS_W
    in_d = [None] * _NCHUNK
    out_d = [None] * _NCHUNK
    for i in range(min(_NBUF, _NCHUNK)):
        in_d[i] = pltpu.async_copy(
            emb_hbm.at[pl.ds(base + i * _CH_ROWS, _CH_ROWS)],
            buf.at[pl.ds(i * _CH_ROWS, _CH_ROWS)],
            in_sems[i],
        )
    for i in range(_NCHUNK):
        b = i % _NBUF
        in_d[i].wait()
        row0 = b * _CH_ROWS

        @plsc.parallel_loop(0, _CH_E, 16, unroll=8)
        def _(j):
            r = row0 + (j >> 10)
            c = pl.multiple_of(j & (_DIM - 1), 16)
            buf[r, pl.ds(c, 16)] = buf[r, pl.ds(c, 16)] * _SCALE

        out_d[i] = pltpu.async_copy(
            buf.at[pl.ds(row0, _CH_ROWS)],
            out_hbm.at[pl.ds(base + i * _CH_ROWS, _CH_ROWS)],
            out_sems[b],
        )
        nxt = i + _NBUF
        if nxt < _NCHUNK:
            out_d[i].wait()
            in_d[nxt] = pltpu.async_copy(
                emb_hbm.at[pl.ds(base + nxt * _CH_ROWS, _CH_ROWS)],
                buf.at[pl.ds(row0, _CH_ROWS)],
                in_sems[b],
            )
    for i in range(max(0, _NCHUNK - _NBUF), _NCHUNK):
        out_d[i].wait()


def kernel(x, emb):
    del x
    return _sc_scale(emb)


# speedup vs baseline: 1.3633x; 1.0124x over previous
import functools
import jax
import jax.numpy as jnp
from jax import lax
from jax.experimental import pallas as pl
from jax.experimental.pallas import tpu as pltpu, tpu_sc as plsc

_SEQ, _DIM = 8192, 1024
_SCALE = _DIM ** (-0.5)
_NC, _NS = 2, 16
_NW = _NC * _NS
_ROWS_W = _SEQ // _NW
_CH_ROWS = 16
_NCHUNK = _ROWS_W // _CH_ROWS
_NIN = 4
_NOUT = 3
_CH_E = _CH_ROWS * _DIM

_mesh = plsc.VectorSubcoreMesh(core_axis_name="c", subcore_axis_name="s")


@functools.partial(
    pl.kernel,
    out_type=jax.ShapeDtypeStruct((_SEQ, _DIM), jnp.float32),
    mesh=_mesh,
    scratch_types=[
        pltpu.VMEM((_NIN * _CH_ROWS, _DIM), jnp.float32),
        pltpu.VMEM((_NOUT * _CH_ROWS, _DIM), jnp.float32),
    ]
    + [pltpu.SemaphoreType.DMA] * (_NIN + _NOUT),
)
def _sc_scale(emb_hbm, out_hbm, ibuf, obuf, *sems):
    in_sems = sems[:_NIN]
    out_sems = sems[_NIN:]
    wid = lax.axis_index("s") * _NC + lax.axis_index("c")
    base = wid * _ROWS_W
    in_d = [None] * _NCHUNK
    out_d = [None] * _NCHUNK
    for i in range(min(_NIN, _NCHUNK)):
        in_d[i] = pltpu.async_copy(
            emb_hbm.at[pl.ds(base + i * _CH_ROWS, _CH_ROWS)],
            ibuf.at[pl.ds(i * _CH_ROWS, _CH_ROWS)],
            in_sems[i],
        )
    for i in range(_NCHUNK):
        bi = i % _NIN
        bo = i % _NOUT
        irow0 = bi * _CH_ROWS
        orow0 = bo * _CH_ROWS
        in_d[i].wait()
        if i >= _NOUT:
            out_d[i - _NOUT].wait()

        @plsc.parallel_loop(0, _CH_E, 16, unroll=8)
        def _(j):
            ir = irow0 + (j >> 10)
            orr = orow0 + (j >> 10)
            c = pl.multiple_of(j & (_DIM - 1), 16)
            obuf[orr, pl.ds(c, 16)] = ibuf[ir, pl.ds(c, 16)] * _SCALE

        out_d[i] = pltpu.async_copy(
            obuf.at[pl.ds(orow0, _CH_ROWS)],
            out_hbm.at[pl.ds(base + i * _CH_ROWS, _CH_ROWS)],
            out_sems[bo],
        )
        nxt = i + _NIN
        if nxt < _NCHUNK:
            in_d[nxt] = pltpu.async_copy(
                emb_hbm.at[pl.ds(base + nxt * _CH_ROWS, _CH_ROWS)],
                ibuf.at[pl.ds(irow0, _CH_ROWS)],
                in_sems[bi],
            )
    for i in range(max(0, _NCHUNK - _NOUT), _NCHUNK):
        out_d[i].wait()


def kernel(x, emb):
    del x
    return _sc_scale(emb)


# speedup vs baseline: 1.4176x; 1.0398x over previous
import functools
import jax
import jax.numpy as jnp
from jax import lax
from jax.experimental import pallas as pl
from jax.experimental.pallas import tpu as pltpu, tpu_sc as plsc

_SEQ, _DIM = 8192, 1024
_SCALE = _DIM ** (-0.5)
_NC, _NS = 2, 16
_NW = _NC * _NS
_ROWS_W = _SEQ // _NW
_CH_ROWS = 16
_NCHUNK = _ROWS_W // _CH_ROWS
_NBUF = 7
_CH_E = _CH_ROWS * _DIM

_mesh = plsc.VectorSubcoreMesh(core_axis_name="c", subcore_axis_name="s")


@functools.partial(
    pl.kernel,
    out_type=jax.ShapeDtypeStruct((_SEQ, _DIM), jnp.float32),
    mesh=_mesh,
    scratch_types=[pltpu.VMEM((_NBUF * _CH_ROWS, _DIM), jnp.float32)]
    + [pltpu.SemaphoreType.DMA] * (2 * _NBUF),
)
def _sc_scale(emb_hbm, out_hbm, buf, *sems):
    in_sems = sems[:_NBUF]
    out_sems = sems[_NBUF:]
    wid = lax.axis_index("s") * _NC + lax.axis_index("c")
    base = wid * _ROWS_W
    in_d = [None] * _NCHUNK
    out_d = [None] * _NCHUNK
    waited = [False] * _NCHUNK
    for i in range(min(_NBUF, _NCHUNK)):
        in_d[i] = pltpu.async_copy(
            emb_hbm.at[pl.ds(base + i * _CH_ROWS, _CH_ROWS)],
            buf.at[pl.ds(i * _CH_ROWS, _CH_ROWS)],
            in_sems[i],
        )
    for i in range(_NCHUNK):
        b = i % _NBUF
        in_d[i].wait()
        row0 = b * _CH_ROWS

        @plsc.parallel_loop(0, _CH_E, 16, unroll=8)
        def _(j):
            r = row0 + (j >> 10)
            c = pl.multiple_of(j & (_DIM - 1), 16)
            buf[r, pl.ds(c, 16)] = buf[r, pl.ds(c, 16)] * _SCALE

        out_d[i] = pltpu.async_copy(
            buf.at[pl.ds(row0, _CH_ROWS)],
            out_hbm.at[pl.ds(base + i * _CH_ROWS, _CH_ROWS)],
            out_sems[b],
        )
        j = i - 1
        if j >= 0 and j + _NBUF < _NCHUNK:
            out_d[j].wait()
            waited[j] = True
            in_d[j + _NBUF] = pltpu.async_copy(
                emb_hbm.at[pl.ds(base + (j + _NBUF) * _CH_ROWS, _CH_ROWS)],
                buf.at[pl.ds((j % _NBUF) * _CH_ROWS, _CH_ROWS)],
                in_sems[j % _NBUF],
            )
    for i in range(_NCHUNK):
        if not waited[i]:
            out_d[i].wait()


def kernel(x, emb):
    del x
    return _sc_scale(emb)
